# baseline (speedup 1.0000x reference)
import functools

import jax
import jax.numpy as jnp
from jax import lax
from jax.experimental import pallas as pl
from jax.experimental.pallas import tpu as pltpu
from jax.experimental.pallas import tpu_sc as plsc

H = 224
W = 224
P = H * W
K = 9
C = 96
N = K * P
NW = 32
NP = N // NW
CH = 112
NCHUNK = NP // CH
NV = C // 16
OCH = 113
PW = 4 * C
PPAD = 50432


def _make_table(inp_pad):
    PB = 512

    def body(x_ref, o_ref):
        j = pl.program_id(0)
        t = x_ref[:, pl.ds(j * PB, PB + 256)].T
        o_ref[...] = jnp.concatenate(
            [t[0:PB], t[1:PB + 1], t[W:PB + W], t[W + 1:PB + W + 1]],
            axis=1).astype(jnp.bfloat16)

    return pl.pallas_call(
        body,
        grid=(P // PB,),
        in_specs=[pl.BlockSpec((C, PPAD), lambda j: (0, 0))],
        out_specs=pl.BlockSpec((PB, PW), lambda j: (j, 0)),
        out_shape=jax.ShapeDtypeStruct((P, PW), jnp.bfloat16),
    )(inp_pad)


def _sc_gather_blend(table, off2d):
    mesh = plsc.VectorSubcoreMesh(core_axis_name="c", subcore_axis_name="s")

    @functools.partial(
        pl.kernel,
        out_type=jax.ShapeDtypeStruct((C * K, P), jnp.float32),
        mesh=mesh,
        scratch_types=[
            pltpu.VMEM((2, CH), jnp.int32),
            pltpu.VMEM((2, 4, CH), jnp.float32),
            pltpu.VMEM((2, 2, CH), jnp.float32),
            pltpu.VMEM((2, CH, PW), jnp.bfloat16),
            pltpu.VMEM((2, C, OCH), jnp.float32),
            pltpu.SemaphoreType.DMA,
            pltpu.SemaphoreType.DMA,
            pltpu.SemaphoreType.DMA,
            pltpu.SemaphoreType.DMA,
            pltpu.SemaphoreType.DMA,
            pltpu.SemaphoreType.DMA,
        ],
        compiler_params=pltpu.CompilerParams(use_tc_tiling_on_sc=False,
                                             needs_layout_passes=False),
    )
    def run(table_hbm, off_hbm, o_hbm, idx_v, wgt_v, off_v, rows_v, out_t,
            gs0, gs1, os0, os1, is0, is1):
        gsem = (gs0, gs1)
        osem = (os0, os1)
        isem = (is0, is1)
        wid = lax.axis_index("s") * 2 + lax.axis_index("c")
        base = wid * NP

        def fetch_iw(ci, b):
            q0 = base + ci * CH
            k = q0 // P
            p0 = q0 - k * P
            pltpu.async_copy(off_hbm.at[2 * k, pl.ds(p0, CH)],
                             off_v.at[b, 0], isem[b])
            pltpu.async_copy(off_hbm.at[2 * k + 1, pl.ds(p0, CH)],
                             off_v.at[b, 1], isem[b])

        def wait_iw(b):
            pltpu.make_async_copy(off_hbm.at[0, pl.ds(0, CH)],
                                  off_v.at[b, 0], isem[b]).wait()
            pltpu.make_async_copy(off_hbm.at[0, pl.ds(0, CH)],
                                  off_v.at[b, 1], isem[b]).wait()

        def compute_iw(ci, b):
            q0 = base + ci * CH
            k = q0 // P
            p0 = q0 - k * P
            ho = p0 // W
            wo0 = p0 - ho * W
            ybs = (ho - 1 + k // 3).astype(jnp.float32)
            xbs = (wo0 - 1 + k % 3).astype(jnp.float32)
            iotf = lax.iota(jnp.int32, 16).astype(jnp.float32)
            fH = float(H - 1)
            fW = float(W - 1)
            one = jnp.float32(1.0)
            zero = jnp.float32(0.0)
            for g in range(CH // 16):
                y = off_v[b, 0, pl.ds(g * 16, 16)] + ybs
                x = off_v[b, 1, pl.ds(g * 16, 16)] + (iotf + (xbs + g * 16))
                ytf = y.astype(jnp.int32).astype(jnp.float32)
                xtf = x.astype(jnp.int32).astype(jnp.float32)
                y0 = ytf - jnp.where(y < ytf, one, zero)
                x0 = xtf - jnp.where(x < xtf, one, zero)
                ly = y - y0
                lx = x - x0
                hy = 1.0 - ly
                hx = 1.0 - lx
                y1 = y0 + 1.0
                x1 = x0 + 1.0
                vy0 = jnp.where((y0 >= 0.0) & (y0 <= fH), one, zero)
                vy1 = jnp.where((y1 >= 0.0) & (y1 <= fH), one, zero)
                vx0 = jnp.where((x0 >= 0.0) & (x0 <= fW), one, zero)
                vx1 = jnp.where((x1 >= 0.0) & (x1 <= fW), one, zero)
                by = jnp.clip(y0, 0.0, float(H - 2))
                bx = jnp.clip(x0, 0.0, float(W - 2))
                y0c = jnp.clip(y0, 0.0, fH)
                y1c = jnp.clip(y1, 0.0, fH)
                x0c = jnp.clip(x0, 0.0, fW)
                x1c = jnp.clip(x1, 0.0, fW)
                wy0 = hy * vy0
                wy1 = ly * vy1
                wx0 = hx * vx0
                wx1 = lx * vx1
                ys0 = (jnp.where(y0c == by, wy0, zero)
                       + jnp.where(y1c == by, wy1, zero))
                ys1 = (jnp.where(y0c == by, zero, wy0)
                       + jnp.where(y1c == by, zero, wy1))
                xs0 = (jnp.where(x0c == bx, wx0, zero)
                       + jnp.where(x1c == bx, wx1, zero))
                xs1 = (jnp.where(x0c == bx, zero, wx0)
                       + jnp.where(x1c == bx, zero, wx1))
                sl = pl.ds(g * 16, 16)
                idx_v[b, sl] = (by * float(W) + bx).astype(jnp.int32)
                wgt_v[b, 0, sl] = ys0 * xs0
                wgt_v[b, 1, sl] = ys0 * xs1
                wgt_v[b, 2, sl] = ys1 * xs0
                wgt_v[b, 3, sl] = ys1 * xs1

        def fire_gather(b):
            pltpu.async_copy(table_hbm.at[idx_v.at[b]], rows_v.at[b], gsem[b])

        def drain_gathers(b):
            pltpu.make_async_copy(table_hbm.at[pl.ds(0, CH)],
                                  rows_v.at[b], gsem[b]).wait()

        def drain_out(b):
            pltpu.make_async_copy(o_hbm.at[pl.ds(0, C), pl.ds(0, CH)],
                                  out_t.at[b, :, pl.ds(0, CH)], osem[b]).wait()

        def blend(b):
            iot = lax.iota(jnp.int32, 16)
            cidx = [(iot * 2 + g * 32, iot * 2 + (g * 32 + 1))
                    for g in range(C // 32)]

            def g_body(g, carry):
                r0 = g * 16
                wv = [wgt_v[b, j, pl.ds(r0, 16)] for j in range(4)]
                for e in range(16):
                    r = r0 + e
                    ridx = jnp.full((16,), 0, jnp.int32) + r
                    ws = [wv[j][e] for j in range(4)]
                    for v in range(C // 32):
                        o0 = v * 32
                        acc_a = None
                        acc_b = None
                        for s in range(4):
                            x = rows_v[b, r, pl.ds(s * C + o0, 32)]
                            pa, pb = plsc.unpack(
                                x, format=plsc.PackFormat.INTERLEAVED)
                            if s == 0:
                                acc_a = pa * ws[0]
                                acc_b = pb * ws[0]
                            else:
                                acc_a = acc_a + pa * ws[s]
                                acc_b = acc_b + pb * ws[s]
                        plsc.store_scatter(out_t.at[b], [cidx[v][0], ridx], acc_a)
                        plsc.store_scatter(out_t.at[b], [cidx[v][1], ridx], acc_b)
                return carry

            lax.fori_loop(0, CH // 16, g_body, 0)

        def fire_out(ci, b):
            q0 = base + ci * CH
            k = q0 // P
            p0 = q0 - k * P
            for c in range(C):
                pltpu.async_copy(out_t.at[b, c, pl.ds(0, CH)],
                                 o_hbm.at[c * K + k, pl.ds(p0, CH)], osem[b])

        fetch_iw(0, 0)
        fetch_iw(1, 1)
        wait_iw(0)
        compute_iw(0, 0)
        fire_gather(0)

        def pair_body(h, carry):
            for b in range(2):
                ci = 2 * h + b

                @pl.when(ci + 1 < NCHUNK)
                def _():
                    wait_iw(b ^ 1)
                    compute_iw(ci + 1, b ^ 1)
                    fire_gather(b ^ 1)

                drain_gathers(b)

                @pl.when(ci >= 2)
                def _():
                    drain_out(b)

                blend(b)
                fire_out(ci, b)

                @pl.when(ci + 2 < NCHUNK)
                def _():
                    fetch_iw(ci + 2, b)
            return carry

        lax.fori_loop(0, NCHUNK // 2, pair_body, 0)
        drain_out(0)
        drain_out(1)

    return run(table, off2d)


def kernel(input, offset):
    inp2d = input.reshape(C, P)
    inp_pad = jnp.pad(inp2d, ((0, 0), (0, PPAD - P)))
    off2d = offset.reshape(2 * K, P)
    table = _make_table(inp_pad)
    out = _sc_gather_blend(table, off2d)
    return out.reshape(1, C * K, P)


# speedup vs baseline: 1.0662x; 1.0662x over previous
import functools

import jax
import jax.numpy as jnp
from jax import lax
from jax.experimental import pallas as pl
from jax.experimental.pallas import tpu as pltpu
from jax.experimental.pallas import tpu_sc as plsc

H = 224
W = 224
P = H * W
K = 9
C = 96
N = K * P
NW = 32
NP = N // NW
CH = 112
NCHUNK = NP // CH
NV = C // 16
OCH = 113
PW = 4 * C
PPAD = 50432


def _make_table(inp_pad):
    PB = 512

    def body(x_ref, o_ref):
        j = pl.program_id(0)
        t = x_ref[:, pl.ds(j * PB, PB + 256)].T
        o_ref[...] = jnp.concatenate(
            [t[0:PB], t[1:PB + 1], t[W:PB + W], t[W + 1:PB + W + 1]],
            axis=1).astype(jnp.bfloat16)

    return pl.pallas_call(
        body,
        grid=(P // PB,),
        in_specs=[pl.BlockSpec((C, PPAD), lambda j: (0, 0))],
        out_specs=pl.BlockSpec((PB, PW), lambda j: (j, 0)),
        out_shape=jax.ShapeDtypeStruct((P, PW), jnp.bfloat16),
    )(inp_pad)


def _sc_gather_blend(table, off2d):
    mesh = plsc.VectorSubcoreMesh(core_axis_name="c", subcore_axis_name="s")

    @functools.partial(
        pl.kernel,
        out_type=jax.ShapeDtypeStruct((C * K, P), jnp.float32),
        mesh=mesh,
        scratch_types=[
            pltpu.VMEM((2, CH), jnp.int32),
            pltpu.VMEM((2, 4, CH), jnp.float32),
            pltpu.VMEM((2, 2, CH), jnp.float32),
            pltpu.VMEM((2, CH, PW), jnp.bfloat16),
            pltpu.VMEM((2, C, OCH), jnp.float32),
            pltpu.SemaphoreType.DMA,
            pltpu.SemaphoreType.DMA,
            pltpu.SemaphoreType.DMA,
            pltpu.SemaphoreType.DMA,
            pltpu.SemaphoreType.DMA,
            pltpu.SemaphoreType.DMA,
        ],
        compiler_params=pltpu.CompilerParams(use_tc_tiling_on_sc=False,
                                             needs_layout_passes=False),
    )
    def run(table_hbm, off_hbm, o_hbm, idx_v, wgt_v, off_v, rows_v, out_t,
            gs0, gs1, os0, os1, is0, is1):
        gsem = (gs0, gs1)
        osem = (os0, os1)
        isem = (is0, is1)
        wid = lax.axis_index("s") * 2 + lax.axis_index("c")
        base = wid * NP

        def fetch_iw(ci, b):
            q0 = base + ci * CH
            k = q0 // P
            p0 = q0 - k * P
            pltpu.async_copy(off_hbm.at[2 * k, pl.ds(p0, CH)],
                             off_v.at[b, 0], isem[b])
            pltpu.async_copy(off_hbm.at[2 * k + 1, pl.ds(p0, CH)],
                             off_v.at[b, 1], isem[b])

        def wait_iw(b):
            pltpu.make_async_copy(off_hbm.at[0, pl.ds(0, CH)],
                                  off_v.at[b, 0], isem[b]).wait()
            pltpu.make_async_copy(off_hbm.at[0, pl.ds(0, CH)],
                                  off_v.at[b, 1], isem[b]).wait()

        def compute_iw(ci, b):
            q0 = base + ci * CH
            k = q0 // P
            p0 = q0 - k * P
            ho = p0 // W
            wo0 = p0 - ho * W
            ybs = (ho - 1 + k // 3).astype(jnp.float32)
            xbs = (wo0 - 1 + k % 3).astype(jnp.float32)
            iotf = lax.iota(jnp.int32, 16).astype(jnp.float32)
            fH = float(H - 1)
            fW = float(W - 1)
            one = jnp.float32(1.0)
            zero = jnp.float32(0.0)
            for g in range(CH // 16):
                y = off_v[b, 0, pl.ds(g * 16, 16)] + ybs
                x = off_v[b, 1, pl.ds(g * 16, 16)] + (iotf + (xbs + g * 16))
                ytf = y.astype(jnp.int32).astype(jnp.float32)
                xtf = x.astype(jnp.int32).astype(jnp.float32)
                y0 = ytf - jnp.where(y < ytf, one, zero)
                x0 = xtf - jnp.where(x < xtf, one, zero)
                ly = y - y0
                lx = x - x0
                hy = 1.0 - ly
                hx = 1.0 - lx
                y1 = y0 + 1.0
                x1 = x0 + 1.0
                vy0 = jnp.where((y0 >= 0.0) & (y0 <= fH), one, zero)
                vy1 = jnp.where((y1 >= 0.0) & (y1 <= fH), one, zero)
                vx0 = jnp.where((x0 >= 0.0) & (x0 <= fW), one, zero)
                vx1 = jnp.where((x1 >= 0.0) & (x1 <= fW), one, zero)
                by = jnp.clip(y0, 0.0, float(H - 2))
                bx = jnp.clip(x0, 0.0, float(W - 2))
                y0c = jnp.clip(y0, 0.0, fH)
                y1c = jnp.clip(y1, 0.0, fH)
                x0c = jnp.clip(x0, 0.0, fW)
                x1c = jnp.clip(x1, 0.0, fW)
                wy0 = hy * vy0
                wy1 = ly * vy1
                wx0 = hx * vx0
                wx1 = lx * vx1
                ys0 = (jnp.where(y0c == by, wy0, zero)
                       + jnp.where(y1c == by, wy1, zero))
                ys1 = (jnp.where(y0c == by, zero, wy0)
                       + jnp.where(y1c == by, zero, wy1))
                xs0 = (jnp.where(x0c == bx, wx0, zero)
                       + jnp.where(x1c == bx, wx1, zero))
                xs1 = (jnp.where(x0c == bx, zero, wx0)
                       + jnp.where(x1c == bx, zero, wx1))
                sl = pl.ds(g * 16, 16)
                idx_v[b, sl] = (by * float(W) + bx).astype(jnp.int32)
                wgt_v[b, 0, sl] = ys0 * xs0
                wgt_v[b, 1, sl] = ys0 * xs1
                wgt_v[b, 2, sl] = ys1 * xs0
                wgt_v[b, 3, sl] = ys1 * xs1

        def fire_gather(b):
            pltpu.async_copy(table_hbm.at[idx_v.at[b]], rows_v.at[b], gsem[b])

        def drain_gathers(b):
            pltpu.make_async_copy(table_hbm.at[pl.ds(0, CH)],
                                  rows_v.at[b], gsem[b]).wait()

        def drain_out(b):
            pltpu.make_async_copy(o_hbm.at[pl.ds(0, C), pl.ds(0, CH)],
                                  out_t.at[b, :, pl.ds(0, CH)], osem[b]).wait()

        def blend(b):
            iot = lax.iota(jnp.int32, 16)
            cidx = [(iot * 2 + g * 32, iot * 2 + (g * 32 + 1))
                    for g in range(C // 32)]

            def g_body(g, carry):
                r0 = g * 16
                wv = [wgt_v[b, j, pl.ds(r0, 16)] for j in range(4)]
                for e in range(16):
                    r = r0 + e
                    ridx = jnp.full((16,), 0, jnp.int32) + r
                    ws = [wv[j][e] for j in range(4)]
                    for v in range(C // 32):
                        o0 = v * 32
                        acc_a = None
                        acc_b = None
                        for s in range(4):
                            x = rows_v[b, r, pl.ds(s * C + o0, 32)]
                            pa, pb = plsc.unpack(
                                x, format=plsc.PackFormat.INTERLEAVED)
                            if s == 0:
                                acc_a = pa * ws[0]
                                acc_b = pb * ws[0]
                            else:
                                acc_a = acc_a + pa * ws[s]
                                acc_b = acc_b + pb * ws[s]
                        plsc.store_scatter(out_t.at[b], [cidx[v][0], ridx], acc_a)
                        plsc.store_scatter(out_t.at[b], [cidx[v][1], ridx], acc_b)
                return carry

            lax.fori_loop(0, CH // 16, g_body, 0)

        def fire_out(ci, b):
            q0 = base + ci * CH
            k = q0 // P
            p0 = q0 - k * P
            for c in range(C):
                pltpu.async_copy(out_t.at[b, c, pl.ds(0, CH)],
                                 o_hbm.at[c * K + k, pl.ds(p0, CH)], osem[b])

        fetch_iw(0, 0)
        fetch_iw(1, 1)
        wait_iw(0)
        compute_iw(0, 0)
        fire_gather(0)
        wait_iw(1)
        compute_iw(1, 1)

        def pair_body(h, carry):
            for b in range(2):
                ci = 2 * h + b

                @pl.when(ci + 1 < NCHUNK)
                def _():
                    fire_gather(b ^ 1)

                drain_gathers(b)

                @pl.when(ci + 2 < NCHUNK)
                def _():
                    fetch_iw(ci + 2, b)

                @pl.when(ci >= 2)
                def _():
                    drain_out(b)

                blend(b)
                fire_out(ci, b)

                @pl.when(ci + 2 < NCHUNK)
                def _():
                    wait_iw(b)
                    compute_iw(ci + 2, b)
            return carry

        lax.fori_loop(0, NCHUNK // 2, pair_body, 0)
        drain_out(0)
        drain_out(1)

    return run(table, off2d)


def kernel(input, offset):
    inp2d = input.reshape(C, P)
    inp_pad = jnp.pad(inp2d, ((0, 0), (0, PPAD - P)))
    off2d = offset.reshape(2 * K, P)
    table = _make_table(inp_pad)
    out = _sc_gather_blend(table, off2d)
    return out.reshape(1, C * K, P)
